# baseline (speedup 1.0000x reference)
import functools

import jax
import jax.numpy as jnp
from jax import lax
from jax.experimental import pallas as pl
from jax.experimental.pallas import tpu as pltpu
from jax.experimental.pallas import tpu_sc as plsc

V = 10000
E = 320000
D = 128
H = 64
NREL = 201
RP = 208
NC = 2
NS = 16
CK = 128
NCHUNK = E // CK
VP = 10000
AP = 20000
ROWS_T = AP // NS
DEGROWS_T = VP // NS


DSK = 10
NDSLAB = NCHUNK // DSK


def _deg_body(dst_hbm, degp_hbm, dstbuf, onesbuf, dsem, degsh):
    c = lax.axis_index("c")
    s = lax.axis_index("s")
    w = c * NS + s
    zv = jnp.zeros((16,), jnp.float32)

    def _fill(val):
        def body(r, _):
            onesbuf[r] = val
            return 0
        lax.fori_loop(0, CK, body, 0)

    _fill(zv)
    for k in range(5):
        pltpu.sync_copy(onesbuf.at[pl.ds(0, DEGROWS_T // 5)],
                        degsh.at[pl.ds(s * DEGROWS_T + k * (DEGROWS_T // 5),
                                       DEGROWS_T // 5)])
    _fill(jnp.full((16,), 1.0, jnp.float32))
    plsc.subcore_barrier()

    nloops = (NDSLAB + NC * NS - 1) // (NC * NS)

    def slab_body(k, _):
        t = w + k * (NC * NS)

        @pl.when(t < NDSLAB)
        def _():
            pltpu.sync_copy(dst_hbm.at[pl.ds(t * DSK, DSK)], dstbuf)
            descs = [pltpu.async_copy(onesbuf, degsh.at[dstbuf.at[j]], dsem,
                                      add=True)
                     for j in range(DSK)]
            for d in descs:
                d.wait()
        return 0

    lax.fori_loop(0, nloops, slab_body, 0)
    plsc.subcore_barrier()
    pltpu.sync_copy(degsh.at[pl.ds(s * DEGROWS_T, DEGROWS_T)],
                    degp_hbm.at[c, pl.ds(s * DEGROWS_T, DEGROWS_T)])


def _deg_call(dst2d):
    return pl.kernel(
        _deg_body,
        out_type=jax.ShapeDtypeStruct((NC, VP, 16), jnp.float32),
        mesh=plsc.VectorSubcoreMesh(core_axis_name="c", subcore_axis_name="s"),
        compiler_params=pltpu.CompilerParams(needs_layout_passes=False, use_tc_tiling_on_sc=False),
        scratch_types=[
            pltpu.VMEM((DSK, CK), jnp.int32),
            pltpu.VMEM((CK, 16), jnp.float32),
            pltpu.SemaphoreType.DMA,
            pltpu.VMEM_SHARED((VP, 16), jnp.float32),
        ],
    )(dst2d)


def _prep_body(degp, x, relr, looprel, wrel, x2, rel2, n_out, relout):
    deg = degp[0, 0:V, 0] + degp[1, 0:V, 0]
    n = jnp.where(deg > 0.0, lax.rsqrt(jnp.maximum(deg, 1e-30)), 0.0)
    xs = x[...] * n[:, None]
    x2[0:V, :] = xs[:, :H]
    x2[V:2 * V, :] = xs[:, H:]
    relc = jnp.concatenate([relr[...], looprel[...]], axis=0)
    rel2[0:NREL, :] = relc[:, :H]
    rel2[RP:RP + NREL, :] = relc[:, H:]
    n_out[...] = n
    relout[...] = (relc @ wrel[...])[:, D - 1]


def _prep_call(degp, x, rel_repr, loop_rel, w_rel):
    return pl.pallas_call(
        _prep_body,
        out_shape=(
            jax.ShapeDtypeStruct((2 * V, H), jnp.float32),
            jax.ShapeDtypeStruct((2 * RP, H), jnp.float32),
            jax.ShapeDtypeStruct((V,), jnp.float32),
            jax.ShapeDtypeStruct((NREL,), jnp.float32),
        ),
    )(degp, x, rel_repr, loop_rel, w_rel)


SK = 5
SE = SK * CK
NSUPER = NCHUNK // SK


def _edge_body(ints_hbm, rel2_hbm, x2_hbm, out_hbm,
               intsB, sbufB, aibufB,
               xbuf0, xbuf1, tbuf0, tbuf1, relbuf,
               gsem0, gsem1, ssem0, ssem1, accsh):
    c = lax.axis_index("c")
    s = lax.axis_index("s")
    zv = jnp.zeros((16,), jnp.float32)
    xbufs = (xbuf0, xbuf1)
    tbufs = (tbuf0, tbuf1)
    gsems = (gsem0, gsem1)
    ssems = (ssem0, ssem1)

    pltpu.sync_copy(rel2_hbm.at[pl.ds(c * RP, NREL)], relbuf)

    def zrow(r, _):
        for j in range(H // 16):
            tbuf0[r, pl.ds(j * 16, 16)] = zv
        return 0

    lax.fori_loop(0, CK, zrow, 0)
    for k in range(10):
        pltpu.sync_copy(tbuf0.at[pl.ds(0, ROWS_T // 10)],
                        accsh.at[pl.ds(s * ROWS_T + k * (ROWS_T // 10),
                                       ROWS_T // 10)])
    plsc.subcore_barrier()

    nloops = (NSUPER + NS - 1) // NS

    def super_body(k, _):
        u = s + k * NS

        @pl.when(u < NSUPER)
        def _():
            base = u * SK
            pltpu.sync_copy(ints_hbm.at[pl.ds(base, SK)], intsB)
            shift = jnp.broadcast_to(c * V, (16,)).astype(jnp.int32)
            for row in range(SK):
                for g in range(CK // 16):
                    sl = pl.ds(g * 16, 16)
                    sbufB[row, sl] = intsB[row, 0, sl] + shift
                    aibufB[row, sl] = intsB[row, 3, sl] * V + intsB[row, 1, sl]

            gd = [None, None]
            sd = [None, None]
            gd[0] = pltpu.async_copy(x2_hbm.at[sbufB.at[0]], xbufs[0], gsems[0])
            for j in range(SK):
                if j + 1 < SK:
                    gd[(j + 1) % 2] = pltpu.async_copy(
                        x2_hbm.at[sbufB.at[j + 1]], xbufs[(j + 1) % 2],
                        gsems[(j + 1) % 2])
                gd[j % 2].wait()
                if j >= 2:
                    sd[j % 2].wait()
                xb = xbufs[j % 2]
                tb = tbufs[j % 2]

                @plsc.parallel_loop(0, CK // 16, unroll=4)
                def _group16(g, j=j, xb=xb, tb=tb):
                    tyv = intsB[j, 2, pl.ds(g * 16, 16)]
                    for l in range(16):
                        ty = tyv[l]
                        e = g * 16 + l
                        for j4 in range(H // 16):
                            sl = pl.ds(j4 * 16, 16)
                            tb[e, sl] = xb[e, sl] * relbuf[ty, sl]
                sd[j % 2] = pltpu.async_copy(tb, accsh.at[aibufB.at[j]],
                                             ssems[j % 2], add=True)
            sd[0].wait()
            sd[1].wait()
        return 0

    lax.fori_loop(0, nloops, super_body, 0)
    plsc.subcore_barrier()
    pltpu.sync_copy(accsh.at[pl.ds(s * ROWS_T, ROWS_T)],
                    out_hbm.at[c, pl.ds(s * ROWS_T, ROWS_T)])


def _edge_call(ints, rel2, x2):
    return pl.kernel(
        _edge_body,
        out_type=jax.ShapeDtypeStruct((NC, AP, H), jnp.float32),
        mesh=plsc.VectorSubcoreMesh(core_axis_name="c", subcore_axis_name="s"),
        compiler_params=pltpu.CompilerParams(needs_layout_passes=False, use_tc_tiling_on_sc=False),
        scratch_types=[
            pltpu.VMEM((SK, 4, CK), jnp.int32),
            pltpu.VMEM((SK, CK), jnp.int32),
            pltpu.VMEM((SK, CK), jnp.int32),
            pltpu.VMEM((CK, H), jnp.float32),
            pltpu.VMEM((CK, H), jnp.float32),
            pltpu.VMEM((CK, H), jnp.float32),
            pltpu.VMEM((CK, H), jnp.float32),
            pltpu.VMEM((NREL, H), jnp.float32),
            pltpu.SemaphoreType.DMA,
            pltpu.SemaphoreType.DMA,
            pltpu.SemaphoreType.DMA,
            pltpu.SemaphoreType.DMA,
            pltpu.VMEM_SHARED((AP, H), jnp.float32),
        ],
    )(ints, rel2, x2)


def _final_body(hpre, n, x, looprel, w, bias, gamma, beta, h_out):
    p0 = jnp.concatenate([hpre[0, 0:V, :], hpre[1, 0:V, :]], axis=1)
    p1 = jnp.concatenate([hpre[0, V:2 * V, :], hpre[1, V:2 * V, :]], axis=1)
    wa = w[...]
    sl = (x[...] * looprel[...]) @ wa[2]
    acc = p0 @ wa[0] + p1 @ wa[1]
    h = (acc * n[...][:, None] + sl) / 3.0 + bias[...]
    mu = jnp.mean(h, axis=0)
    var = jnp.mean((h - mu) ** 2, axis=0)
    h_out[...] = (h - mu) * lax.rsqrt(var + 1e-5) * gamma[...] + beta[...]


def _final_call(hpre, n, x, loop_rel, w, bias, gamma, beta):
    return pl.pallas_call(
        _final_body,
        out_shape=jax.ShapeDtypeStruct((V, D), jnp.float32),
    )(hpre, n, x, loop_rel, w, bias, gamma, beta)


@jax.jit
def kernel(x, rel_repr, edge_index, edge_type, edge_dir, w, w_rel, loop_rel,
           bias, gamma, beta):
    src = edge_index[0]
    dst = edge_index[1]
    degp = _deg_call(dst.reshape(NCHUNK, CK))
    x2, rel2, n, rel_out = _prep_call(degp, x, rel_repr, loop_rel, w_rel)
    ints = jnp.stack([src.reshape(NCHUNK, CK), dst.reshape(NCHUNK, CK),
                      edge_type.reshape(NCHUNK, CK), edge_dir.reshape(NCHUNK, CK)],
                     axis=1)
    hpre = _edge_call(ints, rel2, x2)
    h = _final_call(hpre, n, x, loop_rel, w, bias, gamma, beta)
    return h, rel_out


# speedup vs baseline: 1.0154x; 1.0154x over previous
import functools

import jax
import jax.numpy as jnp
from jax import lax
from jax.experimental import pallas as pl
from jax.experimental.pallas import tpu as pltpu
from jax.experimental.pallas import tpu_sc as plsc

V = 10000
E = 320000
D = 128
H = 64
NREL = 201
RP = 208
NC = 2
NS = 16
CK = 128
NCHUNK = E // CK
VP = 10000
AP = 20000
ROWS_T = AP // NS
DEGROWS_T = VP // NS


DSK = 10
NDSLAB = NCHUNK // DSK


def _deg_body(dst_hbm, degp_hbm, dstbuf, onesbuf, dsem, degsh):
    c = lax.axis_index("c")
    s = lax.axis_index("s")
    w = c * NS + s
    zv = jnp.zeros((16,), jnp.float32)

    def _fill(val):
        def body(r, _):
            onesbuf[r] = val
            return 0
        lax.fori_loop(0, CK, body, 0)

    _fill(zv)
    for k in range(5):
        pltpu.sync_copy(onesbuf.at[pl.ds(0, DEGROWS_T // 5)],
                        degsh.at[pl.ds(s * DEGROWS_T + k * (DEGROWS_T // 5),
                                       DEGROWS_T // 5)])
    _fill(jnp.full((16,), 1.0, jnp.float32))
    plsc.subcore_barrier()

    nloops = (NDSLAB + NC * NS - 1) // (NC * NS)

    def slab_body(k, _):
        t = w + k * (NC * NS)

        @pl.when(t < NDSLAB)
        def _():
            pltpu.sync_copy(dst_hbm.at[pl.ds(t * DSK, DSK)], dstbuf)
            descs = [pltpu.async_copy(onesbuf, degsh.at[dstbuf.at[j]], dsem,
                                      add=True)
                     for j in range(DSK)]
            for d in descs:
                d.wait()
        return 0

    lax.fori_loop(0, nloops, slab_body, 0)
    plsc.subcore_barrier()
    pltpu.sync_copy(degsh.at[pl.ds(s * DEGROWS_T, DEGROWS_T)],
                    degp_hbm.at[c, pl.ds(s * DEGROWS_T, DEGROWS_T)])


def _deg_call(dst2d):
    return pl.kernel(
        _deg_body,
        out_type=jax.ShapeDtypeStruct((NC, VP, 16), jnp.float32),
        mesh=plsc.VectorSubcoreMesh(core_axis_name="c", subcore_axis_name="s"),
        compiler_params=pltpu.CompilerParams(needs_layout_passes=False, use_tc_tiling_on_sc=False),
        scratch_types=[
            pltpu.VMEM((DSK, CK), jnp.int32),
            pltpu.VMEM((CK, 16), jnp.float32),
            pltpu.SemaphoreType.DMA,
            pltpu.VMEM_SHARED((VP, 16), jnp.float32),
        ],
    )(dst2d)


def _prep_body(degp, x, relr, looprel, wrel, x2, rel2, n_out, relout):
    deg = degp[0, 0:V, 0] + degp[1, 0:V, 0]
    n = jnp.where(deg > 0.0, lax.rsqrt(jnp.maximum(deg, 1e-30)), 0.0)
    xs = x[...] * n[:, None]
    x2[0:V, :] = xs[:, :H]
    x2[V:2 * V, :] = xs[:, H:]
    relc = jnp.concatenate([relr[...], looprel[...]], axis=0)
    rel2[0:NREL, :] = relc[:, :H]
    rel2[RP:RP + NREL, :] = relc[:, H:]
    n_out[...] = n
    relout[...] = (relc @ wrel[...])[:, D - 1]


def _prep_call(degp, x, rel_repr, loop_rel, w_rel):
    return pl.pallas_call(
        _prep_body,
        out_shape=(
            jax.ShapeDtypeStruct((2 * V, H), jnp.float32),
            jax.ShapeDtypeStruct((2 * RP, H), jnp.float32),
            jax.ShapeDtypeStruct((V,), jnp.float32),
            jax.ShapeDtypeStruct((NREL,), jnp.float32),
        ),
    )(degp, x, rel_repr, loop_rel, w_rel)


SK = 5
SE = SK * CK
NSUPER = NCHUNK // SK


def _edge_body(ints_hbm, rel2_hbm, x2_hbm, out_hbm,
               intsB, sbufB, aibufB,
               xbuf0, xbuf1, tbuf0, tbuf1, relbuf,
               gsem0, gsem1, ssem0, ssem1, accsh):
    c = lax.axis_index("c")
    s = lax.axis_index("s")
    zv = jnp.zeros((16,), jnp.float32)
    xbufs = (xbuf0, xbuf1)
    tbufs = (tbuf0, tbuf1)
    gsems = (gsem0, gsem1)
    ssems = (ssem0, ssem1)

    pltpu.sync_copy(rel2_hbm.at[pl.ds(c * RP, NREL)], relbuf)

    def zrow(r, _):
        for j in range(H // 16):
            tbuf0[r, pl.ds(j * 16, 16)] = zv
        return 0

    lax.fori_loop(0, CK, zrow, 0)
    for k in range(10):
        pltpu.sync_copy(tbuf0.at[pl.ds(0, ROWS_T // 10)],
                        accsh.at[pl.ds(s * ROWS_T + k * (ROWS_T // 10),
                                       ROWS_T // 10)])
    plsc.subcore_barrier()

    nloops = (NSUPER + NS - 1) // NS

    def super_body(k, _):
        u = s + k * NS

        @pl.when(u < NSUPER)
        def _():
            base = u * SK
            pltpu.sync_copy(ints_hbm.at[pl.ds(base, SK)], intsB)
            shift = jnp.broadcast_to(c * V, (16,)).astype(jnp.int32)
            for row in range(SK):
                for g in range(CK // 16):
                    sl = pl.ds(g * 16, 16)
                    sbufB[row, sl] = intsB[row, 0, sl] + shift
                    aibufB[row, sl] = intsB[row, 3, sl] * V + intsB[row, 1, sl]

            gd = [None, None]
            sd = [None, None]
            gd[0] = pltpu.async_copy(x2_hbm.at[sbufB.at[0]], xbufs[0], gsems[0])
            for j in range(SK):
                if j + 1 < SK:
                    gd[(j + 1) % 2] = pltpu.async_copy(
                        x2_hbm.at[sbufB.at[j + 1]], xbufs[(j + 1) % 2],
                        gsems[(j + 1) % 2])
                gd[j % 2].wait()
                if j >= 2:
                    sd[j % 2].wait()
                xb = xbufs[j % 2]
                tb = tbufs[j % 2]

                @plsc.parallel_loop(0, CK // 16, unroll=2)
                def _group16(g, j=j, xb=xb, tb=tb):
                    tyv = intsB[j, 2, pl.ds(g * 16, 16)]
                    for l in range(16):
                        ty = tyv[l]
                        e = g * 16 + l
                        for j4 in range(H // 16):
                            sl = pl.ds(j4 * 16, 16)
                            tb[e, sl] = xb[e, sl] * relbuf[ty, sl]
                sd[j % 2] = pltpu.async_copy(tb, accsh.at[aibufB.at[j]],
                                             ssems[j % 2], add=True)
            sd[0].wait()
            sd[1].wait()
        return 0

    lax.fori_loop(0, nloops, super_body, 0)
    plsc.subcore_barrier()
    pltpu.sync_copy(accsh.at[pl.ds(s * ROWS_T, ROWS_T)],
                    out_hbm.at[c, pl.ds(s * ROWS_T, ROWS_T)])


def _edge_call(ints, rel2, x2):
    return pl.kernel(
        _edge_body,
        out_type=jax.ShapeDtypeStruct((NC, AP, H), jnp.float32),
        mesh=plsc.VectorSubcoreMesh(core_axis_name="c", subcore_axis_name="s"),
        compiler_params=pltpu.CompilerParams(needs_layout_passes=False, use_tc_tiling_on_sc=False),
        scratch_types=[
            pltpu.VMEM((SK, 4, CK), jnp.int32),
            pltpu.VMEM((SK, CK), jnp.int32),
            pltpu.VMEM((SK, CK), jnp.int32),
            pltpu.VMEM((CK, H), jnp.float32),
            pltpu.VMEM((CK, H), jnp.float32),
            pltpu.VMEM((CK, H), jnp.float32),
            pltpu.VMEM((CK, H), jnp.float32),
            pltpu.VMEM((NREL, H), jnp.float32),
            pltpu.SemaphoreType.DMA,
            pltpu.SemaphoreType.DMA,
            pltpu.SemaphoreType.DMA,
            pltpu.SemaphoreType.DMA,
            pltpu.VMEM_SHARED((AP, H), jnp.float32),
        ],
    )(ints, rel2, x2)


def _final_body(hpre, n, x, looprel, w, bias, gamma, beta, h_out):
    p0 = jnp.concatenate([hpre[0, 0:V, :], hpre[1, 0:V, :]], axis=1)
    p1 = jnp.concatenate([hpre[0, V:2 * V, :], hpre[1, V:2 * V, :]], axis=1)
    wa = w[...]
    sl = (x[...] * looprel[...]) @ wa[2]
    acc = p0 @ wa[0] + p1 @ wa[1]
    h = (acc * n[...][:, None] + sl) / 3.0 + bias[...]
    mu = jnp.mean(h, axis=0)
    var = jnp.mean((h - mu) ** 2, axis=0)
    h_out[...] = (h - mu) * lax.rsqrt(var + 1e-5) * gamma[...] + beta[...]


def _final_call(hpre, n, x, loop_rel, w, bias, gamma, beta):
    return pl.pallas_call(
        _final_body,
        out_shape=jax.ShapeDtypeStruct((V, D), jnp.float32),
    )(hpre, n, x, loop_rel, w, bias, gamma, beta)


@jax.jit
def kernel(x, rel_repr, edge_index, edge_type, edge_dir, w, w_rel, loop_rel,
           bias, gamma, beta):
    src = edge_index[0]
    dst = edge_index[1]
    degp = _deg_call(dst.reshape(NCHUNK, CK))
    x2, rel2, n, rel_out = _prep_call(degp, x, rel_repr, loop_rel, w_rel)
    ints = jnp.stack([src.reshape(NCHUNK, CK), dst.reshape(NCHUNK, CK),
                      edge_type.reshape(NCHUNK, CK), edge_dir.reshape(NCHUNK, CK)],
                     axis=1)
    hpre = _edge_call(ints, rel2, x2)
    h = _final_call(hpre, n, x, loop_rel, w, bias, gamma, beta)
    return h, rel_out


# speedup vs baseline: 1.0160x; 1.0006x over previous
import jax
import jax.numpy as jnp
from jax import lax
from jax.experimental import pallas as pl
from jax.experimental.pallas import tpu as pltpu
from jax.experimental.pallas import tpu_sc as plsc

V = 10000
E = 320000
D = 128
H = 64
NREL = 201
RP = 208
NC = 2
NS = 16
CK = 128
NCHUNK = E // CK
VP = 10000
AP = 20000
ROWS_T = AP // NS
DEGROWS_T = VP // NS


DSK = 10
NDSLAB = NCHUNK // DSK


def _deg_body(dst_hbm, degp_hbm, dstbuf, onesbuf, dsem, degsh):
    c = lax.axis_index("c")
    s = lax.axis_index("s")
    w = c * NS + s
    zv = jnp.zeros((16,), jnp.float32)

    def _fill(val):
        def body(r, _):
            onesbuf[r] = val
            return 0
        lax.fori_loop(0, CK, body, 0)

    _fill(zv)
    for k in range(5):
        pltpu.sync_copy(onesbuf.at[pl.ds(0, DEGROWS_T // 5)],
                        degsh.at[pl.ds(s * DEGROWS_T + k * (DEGROWS_T // 5),
                                       DEGROWS_T // 5)])
    _fill(jnp.full((16,), 1.0, jnp.float32))
    plsc.subcore_barrier()

    nloops = (NDSLAB + NC * NS - 1) // (NC * NS)

    def slab_body(k, _):
        t = w + k * (NC * NS)

        @pl.when(t < NDSLAB)
        def _():
            pltpu.sync_copy(dst_hbm.at[pl.ds(t * DSK, DSK)], dstbuf)
            descs = [pltpu.async_copy(onesbuf, degsh.at[dstbuf.at[j]], dsem,
                                      add=True)
                     for j in range(DSK)]
            for d in descs:
                d.wait()
        return 0

    lax.fori_loop(0, nloops, slab_body, 0)
    plsc.subcore_barrier()
    pltpu.sync_copy(degsh.at[pl.ds(s * DEGROWS_T, DEGROWS_T)],
                    degp_hbm.at[c, pl.ds(s * DEGROWS_T, DEGROWS_T)])


def _deg_call(dst2d):
    return pl.kernel(
        _deg_body,
        out_type=jax.ShapeDtypeStruct((NC, VP, 16), jnp.float32),
        mesh=plsc.VectorSubcoreMesh(core_axis_name="c", subcore_axis_name="s"),
        compiler_params=pltpu.CompilerParams(needs_layout_passes=False, use_tc_tiling_on_sc=False),
        scratch_types=[
            pltpu.VMEM((DSK, CK), jnp.int32),
            pltpu.VMEM((CK, 16), jnp.float32),
            pltpu.SemaphoreType.DMA,
            pltpu.VMEM_SHARED((VP, 16), jnp.float32),
        ],
    )(dst2d)


def _prep_body(degp, x, relr, looprel, wrel, x2, rel2, n_out, relout):
    deg = degp[0, 0:V, 0] + degp[1, 0:V, 0]
    n = jnp.where(deg > 0.0, lax.rsqrt(jnp.maximum(deg, 1e-30)), 0.0)
    xs = x[...] * n[:, None]
    x2[0:V, :] = xs[:, :H]
    x2[V:2 * V, :] = xs[:, H:]
    relc = jnp.concatenate([relr[...], looprel[...]], axis=0)
    rel2[0:NREL, :] = relc[:, :H]
    rel2[RP:RP + NREL, :] = relc[:, H:]
    n_out[...] = n
    relout[...] = (relc @ wrel[...])[:, D - 1]


def _prep_call(degp, x, rel_repr, loop_rel, w_rel):
    return pl.pallas_call(
        _prep_body,
        out_shape=(
            jax.ShapeDtypeStruct((2 * V, H), jnp.float32),
            jax.ShapeDtypeStruct((2 * RP, H), jnp.float32),
            jax.ShapeDtypeStruct((V,), jnp.float32),
            jax.ShapeDtypeStruct((NREL,), jnp.float32),
        ),
    )(degp, x, rel_repr, loop_rel, w_rel)


SK = 5
SE = SK * CK
NSUPER = NCHUNK // SK


def _edge_body(ints_hbm, rel2_hbm, x2_hbm, out_hbm,
               intsB, sbufB, aibufB,
               xbuf0, xbuf1, tbuf0, tbuf1, relbuf,
               gsem0, gsem1, ssem0, ssem1, accsh):
    c = lax.axis_index("c")
    s = lax.axis_index("s")
    zv = jnp.zeros((16,), jnp.float32)
    xbufs = (xbuf0, xbuf1)
    tbufs = (tbuf0, tbuf1)
    gsems = (gsem0, gsem1)
    ssems = (ssem0, ssem1)

    pltpu.sync_copy(rel2_hbm.at[pl.ds(c * RP, NREL)], relbuf)

    def zrow(r, _):
        for j in range(H // 16):
            tbuf0[r, pl.ds(j * 16, 16)] = zv
        return 0

    lax.fori_loop(0, CK, zrow, 0)
    for k in range(10):
        pltpu.sync_copy(tbuf0.at[pl.ds(0, ROWS_T // 10)],
                        accsh.at[pl.ds(s * ROWS_T + k * (ROWS_T // 10),
                                       ROWS_T // 10)])
    plsc.subcore_barrier()

    nloops = (NSUPER + NS - 1) // NS

    def super_body(k, _):
        u = s + k * NS

        @pl.when(u < NSUPER)
        def _():
            base = u * SK
            pltpu.sync_copy(ints_hbm.at[pl.ds(base, SK)], intsB)
            shift = jnp.broadcast_to(c * V, (16,)).astype(jnp.int32)
            for row in range(SK):
                for g in range(CK // 16):
                    sl = pl.ds(g * 16, 16)
                    sbufB[row, sl] = intsB[row, 0, sl] + shift
                    aibufB[row, sl] = intsB[row, 3, sl] * V + intsB[row, 1, sl]

            gd = [None, None]
            sd = [None, None]
            gd[0] = pltpu.async_copy(x2_hbm.at[sbufB.at[0]], xbufs[0], gsems[0])
            for j in range(SK):
                if j + 1 < SK:
                    gd[(j + 1) % 2] = pltpu.async_copy(
                        x2_hbm.at[sbufB.at[j + 1]], xbufs[(j + 1) % 2],
                        gsems[(j + 1) % 2])
                gd[j % 2].wait()
                if j >= 2:
                    sd[j % 2].wait()
                xb = xbufs[j % 2]
                tb = tbufs[j % 2]

                @plsc.parallel_loop(0, CK // 16, unroll=2)
                def _group16(g, j=j, xb=xb, tb=tb):
                    tyv = intsB[j, 2, pl.ds(g * 16, 16)]
                    for l in range(16):
                        ty = tyv[l]
                        e = g * 16 + l
                        for j4 in range(H // 16):
                            sl = pl.ds(j4 * 16, 16)
                            tb[e, sl] = xb[e, sl] * relbuf[ty, sl]
                sd[j % 2] = pltpu.async_copy(tb, accsh.at[aibufB.at[j]],
                                             ssems[j % 2], add=True)
            sd[0].wait()
            sd[1].wait()
        return 0

    lax.fori_loop(0, nloops, super_body, 0)
    plsc.subcore_barrier()
    pltpu.sync_copy(accsh.at[pl.ds(s * ROWS_T, ROWS_T)],
                    out_hbm.at[c, pl.ds(s * ROWS_T, ROWS_T)])


def _edge_call(ints, rel2, x2):
    return pl.kernel(
        _edge_body,
        out_type=jax.ShapeDtypeStruct((NC, AP, H), jnp.float32),
        mesh=plsc.VectorSubcoreMesh(core_axis_name="c", subcore_axis_name="s"),
        compiler_params=pltpu.CompilerParams(needs_layout_passes=False, use_tc_tiling_on_sc=False),
        scratch_types=[
            pltpu.VMEM((SK, 4, CK), jnp.int32),
            pltpu.VMEM((SK, CK), jnp.int32),
            pltpu.VMEM((SK, CK), jnp.int32),
            pltpu.VMEM((CK, H), jnp.float32),
            pltpu.VMEM((CK, H), jnp.float32),
            pltpu.VMEM((CK, H), jnp.float32),
            pltpu.VMEM((CK, H), jnp.float32),
            pltpu.VMEM((NREL, H), jnp.float32),
            pltpu.SemaphoreType.DMA,
            pltpu.SemaphoreType.DMA,
            pltpu.SemaphoreType.DMA,
            pltpu.SemaphoreType.DMA,
            pltpu.VMEM_SHARED((AP, H), jnp.float32),
        ],
    )(ints, rel2, x2)


def _final_body(hpre, n, x, looprel, w, bias, gamma, beta, h_out):
    p0 = jnp.concatenate([hpre[0, 0:V, :], hpre[1, 0:V, :]], axis=1)
    p1 = jnp.concatenate([hpre[0, V:2 * V, :], hpre[1, V:2 * V, :]], axis=1)
    wa = w[...]
    sl = (x[...] * looprel[...]) @ wa[2]
    acc = p0 @ wa[0] + p1 @ wa[1]
    h = (acc * n[...][:, None] + sl) / 3.0 + bias[...]
    mu = jnp.mean(h, axis=0)
    var = jnp.mean((h - mu) ** 2, axis=0)
    h_out[...] = (h - mu) * lax.rsqrt(var + 1e-5) * gamma[...] + beta[...]


def _final_call(hpre, n, x, loop_rel, w, bias, gamma, beta):
    return pl.pallas_call(
        _final_body,
        out_shape=jax.ShapeDtypeStruct((V, D), jnp.float32),
    )(hpre, n, x, loop_rel, w, bias, gamma, beta)


@jax.jit
def kernel(x, rel_repr, edge_index, edge_type, edge_dir, w, w_rel, loop_rel,
           bias, gamma, beta):
    src = edge_index[0]
    dst = edge_index[1]
    degp = _deg_call(dst.reshape(NCHUNK, CK))
    x2, rel2, n, rel_out = _prep_call(degp, x, rel_repr, loop_rel, w_rel)
    ints = jnp.stack([src.reshape(NCHUNK, CK), dst.reshape(NCHUNK, CK),
                      edge_type.reshape(NCHUNK, CK), edge_dir.reshape(NCHUNK, CK)],
                     axis=1)
    hpre = _edge_call(ints, rel2, x2)
    h = _final_call(hpre, n, x, loop_rel, w, bias, gamma, beta)
    return h, rel_out
